# Initial kernel scaffold; baseline (speedup 1.0000x reference)
import jax
import jax.numpy as jnp
from jax.experimental import pallas as pl


def kernel(x, scale, per_expert_scale, W_proj):
    raise NotImplementedError("write your pallas kernel here")


# speedup vs baseline: 1.4689x; 1.4689x over previous
import jax
import jax.numpy as jnp
from jax.experimental import pallas as pl

HIDDEN = 2048
NUM_EXPERTS = 16
TOP_K = 2
EPS = 1e-6
TOKENS = 8192

BLOCK_T = 512


def _router_block(x_ref, scale_ref, pes_ref, w_ref, idx_ref, wgt_ref):
    xb = x_ref[...]
    ms = jnp.mean(xb * xb, axis=-1, keepdims=True)
    y = xb * jax.lax.rsqrt(ms + EPS)
    y = y * scale_ref[...]
    y = y * (HIDDEN ** -0.5)
    logits = jax.lax.dot_general(
        y, w_ref[...],
        dimension_numbers=(((1,), (1,)), ((), ())),
        preferred_element_type=jnp.float32,
    )

    m = jnp.max(logits, axis=-1, keepdims=True)
    e = jnp.exp(logits - m)
    p = e / jnp.sum(e, axis=-1, keepdims=True)

    cols = jax.lax.broadcasted_iota(jnp.int32, p.shape, 1)
    m1 = jnp.max(p, axis=-1, keepdims=True)
    i1 = jnp.min(jnp.where(p == m1, cols, NUM_EXPERTS), axis=-1, keepdims=True)
    pm = jnp.where(cols == i1, -1.0, p)
    m2 = jnp.max(pm, axis=-1, keepdims=True)
    i2 = jnp.min(jnp.where(pm == m2, cols, NUM_EXPERTS), axis=-1, keepdims=True)

    s = m1 + m2
    pes = pes_ref[...]
    g1 = jnp.sum(jnp.where(cols == i1, pes, 0.0), axis=-1, keepdims=True)
    g2 = jnp.sum(jnp.where(cols == i2, pes, 0.0), axis=-1, keepdims=True)
    w1 = (m1 / s) * g1
    w2 = (m2 / s) * g2

    idx_ref[...] = jnp.concatenate([i1, i2], axis=-1)
    wgt_ref[...] = jnp.concatenate([w1, w2], axis=-1)


@jax.jit
def kernel(x, scale, per_expert_scale, W_proj):
    grid = (TOKENS // BLOCK_T,)
    idx, wgt = pl.pallas_call(
        _router_block,
        grid=grid,
        in_specs=[
            pl.BlockSpec((BLOCK_T, HIDDEN), lambda i: (i, 0)),
            pl.BlockSpec((1, HIDDEN), lambda i: (0, 0)),
            pl.BlockSpec((1, NUM_EXPERTS), lambda i: (0, 0)),
            pl.BlockSpec((NUM_EXPERTS, HIDDEN), lambda i: (0, 0)),
        ],
        out_specs=[
            pl.BlockSpec((BLOCK_T, TOP_K), lambda i: (i, 0)),
            pl.BlockSpec((BLOCK_T, TOP_K), lambda i: (i, 0)),
        ],
        out_shape=[
            jax.ShapeDtypeStruct((TOKENS, TOP_K), jnp.int32),
            jax.ShapeDtypeStruct((TOKENS, TOP_K), jnp.float32),
        ],
    )(x, scale.reshape(1, HIDDEN), per_expert_scale.reshape(1, NUM_EXPERTS),
      W_proj)
    return idx.astype(jnp.int64), wgt
